# baseline (speedup 1.0000x reference)
import functools

import jax
import jax.numpy as jnp
from jax.experimental import pallas as pl
from jax.experimental.pallas import tpu as pltpu

N_MOL = 16384
N_MOTIF = 512
D = 256
B = 8
K = 8
TN = 2048


def _body(x_ref, seg_ref, mbi_ref, motif_ref, w1_ref, wc_ref, b_ref,
          attach_ref, out_ref, ta_ref, tm_ref):
    pid = pl.program_id(0)

    @pl.when(pid == 0)
    def _build_tables():
        mbi = mbi_ref[...]
        srow = jax.lax.broadcasted_iota(jnp.int32, (B, N_MOTIF), 0)
        cmp = (jnp.broadcast_to(mbi, (B, N_MOTIF)) < srow).astype(jnp.int32)
        offsets = jnp.sum(cmp, axis=1, keepdims=True)
        kcol = jax.lax.broadcasted_iota(jnp.int32, (B, K), 1)
        sel = jnp.clip(offsets + kcol, 0, N_MOTIF - 1)
        p = jax.lax.dot(motif_ref[...], w1_ref[...],
                        preferred_element_type=jnp.float32)
        rowid = jax.lax.broadcasted_iota(jnp.int32, (B, N_MOTIF), 1)
        blocks = []
        for k in range(K):
            oh_k = (rowid == sel[:, k:k + 1]).astype(jnp.float32)
            blocks.append(jax.lax.dot(oh_k, p,
                                      preferred_element_type=jnp.float32))
        a = jnp.concatenate(blocks, axis=1)
        b_rep = jnp.concatenate([b_ref[...]] * K, axis=1)
        ta_ref[...] = a + b_rep
        r8 = jax.lax.broadcasted_iota(jnp.int32, (K, K * 4), 0)
        c32 = jax.lax.broadcasted_iota(jnp.int32, (K, K * 4), 1) // 4
        expand = (r8 == c32).astype(jnp.float32)
        tm_ref[...] = jax.lax.dot(attach_ref[...], expand,
                                  preferred_element_type=jnp.float32)

    seg = seg_ref[...]
    lanes = jax.lax.broadcasted_iota(jnp.int32, (TN, B), 1)
    oh = (seg == lanes).astype(jnp.float32)
    arows = jax.lax.dot(oh, ta_ref[...], preferred_element_type=jnp.float32)
    mrows = jax.lax.dot(oh, tm_ref[...], preferred_element_type=jnp.float32)
    acc = jax.lax.dot(x_ref[...], wc_ref[...],
                      preferred_element_type=jnp.float32)
    out_ref[...] = jnp.maximum(acc + arows, 0.0) * mrows


@jax.jit
def kernel(mol_atom_hiddens, mol_batch_indices, motif_atom_hiddens,
           motif_batch_indices, selected_attachments, W, b):
    n = mol_atom_hiddens.shape[0]
    grid = n // TN
    w1 = W[:D, :]
    w2 = W[D:, :]
    wc = jnp.tile(w2, (1, K))
    seg_col = mol_batch_indices.reshape(n, 1)
    mbi_row = motif_batch_indices.reshape(1, N_MOTIF)
    attach_f = selected_attachments.astype(jnp.float32)
    b_row = b.reshape(1, 4)

    out32 = pl.pallas_call(
        _body,
        grid=(grid,),
        in_specs=[
            pl.BlockSpec((TN, D), lambda i: (i, 0)),
            pl.BlockSpec((TN, 1), lambda i: (i, 0)),
            pl.BlockSpec((1, N_MOTIF), lambda i: (0, 0)),
            pl.BlockSpec((N_MOTIF, D), lambda i: (0, 0)),
            pl.BlockSpec((D, 4), lambda i: (0, 0)),
            pl.BlockSpec((D, K * 4), lambda i: (0, 0)),
            pl.BlockSpec((1, 4), lambda i: (0, 0)),
            pl.BlockSpec((B, K), lambda i: (0, 0)),
        ],
        out_specs=pl.BlockSpec((TN, K * 4), lambda i: (i, 0)),
        out_shape=jax.ShapeDtypeStruct((n, K * 4), jnp.float32),
        scratch_shapes=[
            pltpu.VMEM((B, K * 4), jnp.float32),
            pltpu.VMEM((B, K * 4), jnp.float32),
        ],
        compiler_params=pltpu.CompilerParams(
            dimension_semantics=("arbitrary",),
        ),
    )(mol_atom_hiddens, seg_col, mbi_row, motif_atom_hiddens,
      w1, wc, b_row, attach_f)

    return out32.reshape(n, K, 4)


# speedup vs baseline: 1.0980x; 1.0980x over previous
import functools

import jax
import jax.numpy as jnp
from jax.experimental import pallas as pl
from jax.experimental.pallas import tpu as pltpu

N_MOL = 16384
N_MOTIF = 512
D = 256
B = 8
K = 8
TN = 8192


def _body(x_ref, seg_ref, mbi_ref, motif_ref, w1_ref, wc_ref, b_ref,
          attach_ref, out_ref, ta_ref, tm_ref):
    pid = pl.program_id(0)

    @pl.when(pid == 0)
    def _build_tables():
        mbi = mbi_ref[...]
        srow = jax.lax.broadcasted_iota(jnp.int32, (B, N_MOTIF), 0)
        cmp = (jnp.broadcast_to(mbi, (B, N_MOTIF)) < srow).astype(jnp.int32)
        offsets = jnp.sum(cmp, axis=1, keepdims=True)
        kcol = jax.lax.broadcasted_iota(jnp.int32, (B, K), 1)
        sel = jnp.clip(offsets + kcol, 0, N_MOTIF - 1)
        p = jax.lax.dot(motif_ref[...], w1_ref[...],
                        preferred_element_type=jnp.float32)
        rowid = jax.lax.broadcasted_iota(jnp.int32, (B, N_MOTIF), 1)
        blocks = []
        for k in range(K):
            oh_k = (rowid == sel[:, k:k + 1]).astype(jnp.float32)
            blocks.append(jax.lax.dot(oh_k, p,
                                      preferred_element_type=jnp.float32))
        a = jnp.concatenate(blocks, axis=1)
        b_rep = jnp.concatenate([b_ref[...]] * K, axis=1)
        ta_ref[...] = a + b_rep
        r8 = jax.lax.broadcasted_iota(jnp.int32, (K, K * 4), 0)
        c32 = jax.lax.broadcasted_iota(jnp.int32, (K, K * 4), 1) // 4
        expand = (r8 == c32).astype(jnp.float32)
        tm_ref[...] = jax.lax.dot(attach_ref[...], expand,
                                  preferred_element_type=jnp.float32)

    seg = seg_ref[...]
    lanes = jax.lax.broadcasted_iota(jnp.int32, (TN, B), 1)
    oh = (seg == lanes).astype(jnp.float32)
    arows = jax.lax.dot(oh, ta_ref[...], preferred_element_type=jnp.float32)
    mrows = jax.lax.dot(oh, tm_ref[...], preferred_element_type=jnp.float32)
    acc = jax.lax.dot(x_ref[...], wc_ref[...],
                      preferred_element_type=jnp.float32)
    out_ref[...] = jnp.maximum(acc + arows, 0.0) * mrows


@jax.jit
def kernel(mol_atom_hiddens, mol_batch_indices, motif_atom_hiddens,
           motif_batch_indices, selected_attachments, W, b):
    n = mol_atom_hiddens.shape[0]
    grid = n // TN
    w1 = W[:D, :]
    w2 = W[D:, :]
    wc = jnp.tile(w2, (1, K))
    seg_col = mol_batch_indices.reshape(n, 1)
    mbi_row = motif_batch_indices.reshape(1, N_MOTIF)
    attach_f = selected_attachments.astype(jnp.float32)
    b_row = b.reshape(1, 4)

    out32 = pl.pallas_call(
        _body,
        grid=(grid,),
        in_specs=[
            pl.BlockSpec((TN, D), lambda i: (i, 0)),
            pl.BlockSpec((TN, 1), lambda i: (i, 0)),
            pl.BlockSpec((1, N_MOTIF), lambda i: (0, 0)),
            pl.BlockSpec((N_MOTIF, D), lambda i: (0, 0)),
            pl.BlockSpec((D, 4), lambda i: (0, 0)),
            pl.BlockSpec((D, K * 4), lambda i: (0, 0)),
            pl.BlockSpec((1, 4), lambda i: (0, 0)),
            pl.BlockSpec((B, K), lambda i: (0, 0)),
        ],
        out_specs=pl.BlockSpec((TN, K * 4), lambda i: (i, 0)),
        out_shape=jax.ShapeDtypeStruct((n, K * 4), jnp.float32),
        scratch_shapes=[
            pltpu.VMEM((B, K * 4), jnp.float32),
            pltpu.VMEM((B, K * 4), jnp.float32),
        ],
        compiler_params=pltpu.CompilerParams(
            dimension_semantics=("arbitrary",),
        ),
    )(mol_atom_hiddens, seg_col, mbi_row, motif_atom_hiddens,
      w1, wc, b_row, attach_f)

    return out32.reshape(n, K, 4)
